# Initial kernel scaffold; baseline (speedup 1.0000x reference)
import jax
import jax.numpy as jnp
from jax.experimental import pallas as pl


def kernel(v_emb, a_emb):
    raise NotImplementedError("write your pallas kernel here")


# speedup vs baseline: 1.8730x; 1.8730x over previous
import functools

import jax
import jax.numpy as jnp
from jax import lax
from jax.experimental import pallas as pl
from jax.experimental.pallas import tpu as pltpu
from jax.experimental.pallas import tpu_sc as plsc

_TEMP = 0.07
_B = 4096
_D = 16
_K = 5
_NC, _NS, _L = 2, 16, 16
_NW = _NC * _NS
_CW = _B // _NW
_CH = 128
_NCH = _B // _CH
_G = _CW // _L
_RB = 256
_NEG = -3.0e38


def _norm_rows(x):
    return x * lax.rsqrt(jnp.maximum(jnp.sum(x * x, axis=1, keepdims=True),
                                     1e-24))


def _sim_body(v_blk_ref, a_blk_ref, v_all_ref, a_all_ref,
              sim_ref, simt_ref, pos_ref):
    i = pl.program_id(0)
    vb = _norm_rows(v_blk_ref[...])
    ab = _norm_rows(a_blk_ref[...])
    va = _norm_rows(v_all_ref[...])
    aa = _norm_rows(a_all_ref[...])
    row_ids = i * _RB + lax.broadcasted_iota(jnp.int32, (_RB, _B), 0)
    col_ids = lax.broadcasted_iota(jnp.int32, (_RB, _B), 1)
    diag = row_ids == col_ids
    sim = lax.dot_general(vb, aa, (((1,), (1,)), ((), ())),
                          preferred_element_type=jnp.float32) * (1.0 / _TEMP)
    simt = lax.dot_general(ab, va, (((1,), (1,)), ((), ())),
                           preferred_element_type=jnp.float32) * (1.0 / _TEMP)
    sim_ref[...] = jnp.where(diag, _NEG, sim)
    simt_ref[...] = jnp.where(diag, _NEG, simt)
    pos_ref[...] = (jnp.sum(vb * ab, axis=1) * (1.0 / _TEMP)).reshape(1, 1, _RB)


def _compute_sim(v_emb, a_emb):
    return pl.pallas_call(
        _sim_body,
        grid=(_B // _RB,),
        in_specs=[
            pl.BlockSpec((_RB, _D), lambda i: (i, 0)),
            pl.BlockSpec((_RB, _D), lambda i: (i, 0)),
            pl.BlockSpec((_B, _D), lambda i: (0, 0)),
            pl.BlockSpec((_B, _D), lambda i: (0, 0)),
        ],
        out_specs=[
            pl.BlockSpec((_RB, _B), lambda i: (i, 0)),
            pl.BlockSpec((_RB, _B), lambda i: (i, 0)),
            pl.BlockSpec((1, 1, _RB), lambda i: (i, 0, 0)),
        ],
        out_shape=[
            jax.ShapeDtypeStruct((_B, _B), jnp.float32),
            jax.ShapeDtypeStruct((_B, _B), jnp.float32),
            jax.ShapeDtypeStruct((_B // _RB, 1, _RB), jnp.float32),
        ],
    )(v_emb, a_emb, v_emb, a_emb)


def _topk_body(sim_hbm, simt_hbm, pos_hbm, out_hbm,
               chunk_v, acc_v, pos_v, stage_v):
    wid = lax.axis_index("s") * _NC + lax.axis_index("c")
    c0 = wid * _CW
    neg = jnp.full((_L,), _NEG, jnp.float32)
    pltpu.sync_copy(pos_hbm.at[pl.ds(c0, _CW)], pos_v)

    for d, src in ((0, simt_hbm), (1, sim_hbm)):
        for g in range(_G):
            for t in range(_K):
                acc_v[g, t, :] = neg

        def chunk_body(ci, carry):
            j0 = ci * _CH
            pltpu.sync_copy(src.at[pl.ds(j0, _CH), pl.ds(c0, _CW)], chunk_v)
            for g in range(_G):
                ts = tuple(acc_v[g, t, :] for t in range(_K))

                def jbody(j, ts, g=g):
                    x = chunk_v[j, pl.ds(g * _L, _L)]
                    out = []
                    for t in ts:
                        nt = jnp.maximum(t, x)
                        x = jnp.minimum(t, x)
                        out.append(nt)
                    return tuple(out)

                ts = lax.fori_loop(0, _CH, jbody, ts)
                for t in range(_K):
                    acc_v[g, t, :] = ts[t]
            return carry

        lax.fori_loop(0, _NCH, chunk_body, 0)

        for g in range(_G):
            pos = pos_v[pl.ds(g * _L, _L)]
            s = jnp.exp(acc_v[g, 0, :] - pos)
            for t in range(1, _K):
                s = s + jnp.exp(acc_v[g, t, :] - pos)
            stage_v[pl.ds(g * _L, _L)] = s
        pltpu.sync_copy(stage_v, out_hbm.at[pl.ds(d * _B + c0, _CW)])


@functools.cache
def _topk_sc():
    return pl.kernel(
        _topk_body,
        out_type=jax.ShapeDtypeStruct((2 * _B,), jnp.float32),
        mesh=plsc.VectorSubcoreMesh(core_axis_name="c", subcore_axis_name="s",
                                    num_cores=_NC, num_subcores=_NS),
        scratch_types=[
            pltpu.VMEM((_CH, _CW), jnp.float32),
            pltpu.VMEM((_G, _K, _L), jnp.float32),
            pltpu.VMEM((_CW,), jnp.float32),
            pltpu.VMEM((_CW,), jnp.float32),
        ],
    )


def _reduce_body(s_ref, o_ref):
    o_ref[0, 0] = jnp.sum(jnp.log1p(s_ref[...])) * (1.0 / (2 * _B))


def _reduce(s):
    out = pl.pallas_call(
        _reduce_body,
        out_specs=pl.BlockSpec(memory_space=pltpu.SMEM),
        out_shape=jax.ShapeDtypeStruct((1, 1), jnp.float32),
    )(s.reshape(2 * _B // 128, 128))
    return out[0, 0]


def kernel(v_emb, a_emb):
    sim, simt, pos = _compute_sim(v_emb, a_emb)
    s = _topk_sc()(sim, simt, pos.reshape(_B))
    return _reduce(s)


# speedup vs baseline: 3.7128x; 1.9823x over previous
import functools

import jax
import jax.numpy as jnp
from jax import lax
from jax.experimental import pallas as pl
from jax.experimental.pallas import tpu as pltpu
from jax.experimental.pallas import tpu_sc as plsc

_TEMP = 0.07
_B = 4096
_D = 16
_K = 5
_NC, _NS, _L = 2, 16, 16
_NW = _NC * _NS
_CW = _B // _NW
_CH = 128
_NCH = _B // _CH
_G = _CW // _L
_RB = 256
_NEG = -3.0e38


def _norm_rows(x):
    return x * lax.rsqrt(jnp.maximum(jnp.sum(x * x, axis=1, keepdims=True),
                                     1e-24))


def _sim_body(v_blk_ref, a_blk_ref, v_all_ref, a_all_ref,
              sim_ref, simt_ref, pos_ref):
    i = pl.program_id(0)
    vb = _norm_rows(v_blk_ref[...])
    ab = _norm_rows(a_blk_ref[...])
    va = _norm_rows(v_all_ref[...])
    aa = _norm_rows(a_all_ref[...])
    row_ids = i * _RB + lax.broadcasted_iota(jnp.int32, (_RB, _B), 0)
    col_ids = lax.broadcasted_iota(jnp.int32, (_RB, _B), 1)
    diag = row_ids == col_ids
    sim = lax.dot_general(vb, aa, (((1,), (1,)), ((), ())),
                          preferred_element_type=jnp.float32) * (1.0 / _TEMP)
    simt = lax.dot_general(ab, va, (((1,), (1,)), ((), ())),
                           preferred_element_type=jnp.float32) * (1.0 / _TEMP)
    sim_ref[...] = jnp.where(diag, _NEG, sim)
    simt_ref[...] = jnp.where(diag, _NEG, simt)
    pos_ref[...] = (jnp.sum(vb * ab, axis=1) * (1.0 / _TEMP)).reshape(1, 1, _RB)


def _compute_sim(v_emb, a_emb):
    return pl.pallas_call(
        _sim_body,
        grid=(_B // _RB,),
        in_specs=[
            pl.BlockSpec((_RB, _D), lambda i: (i, 0)),
            pl.BlockSpec((_RB, _D), lambda i: (i, 0)),
            pl.BlockSpec((_B, _D), lambda i: (0, 0)),
            pl.BlockSpec((_B, _D), lambda i: (0, 0)),
        ],
        out_specs=[
            pl.BlockSpec((_RB, _B), lambda i: (i, 0)),
            pl.BlockSpec((_RB, _B), lambda i: (i, 0)),
            pl.BlockSpec((1, 1, _RB), lambda i: (i, 0, 0)),
        ],
        out_shape=[
            jax.ShapeDtypeStruct((_B, _B), jnp.float32),
            jax.ShapeDtypeStruct((_B, _B), jnp.float32),
            jax.ShapeDtypeStruct((_B // _RB, 1, _RB), jnp.float32),
        ],
    )(v_emb, a_emb, v_emb, a_emb)


_U = 2
_GI = 4


def _topk_body(sim_hbm, simt_hbm, pos_hbm, out_hbm,
               chunk_v, acc_v, pos_v, stage_v, sem0, sem1):
    wid = lax.axis_index("s") * _NC + lax.axis_index("c")
    c0 = wid * _CW
    neg = jnp.full((_L,), _NEG, jnp.float32)
    pltpu.sync_copy(pos_hbm.at[pl.ds(c0, _CW)], pos_v)
    sems = (sem0, sem1)

    for d, src in ((0, simt_hbm), (1, sim_hbm)):
        for g in range(_G):
            for t in range(_K):
                acc_v[g, t, :] = neg

        for b in range(2):
            pltpu.async_copy(src.at[pl.ds(b * _CH, _CH), pl.ds(c0, _CW)],
                             chunk_v.at[b], sems[b])

        @pl.loop(0, _NCH, step=2)
        def _(ci0, src=src):
            for b in range(2):
                ci = ci0 + b
                pltpu.make_async_copy(
                    src.at[pl.ds(0, _CH), pl.ds(c0, _CW)],
                    chunk_v.at[b], sems[b]).wait()
                for gg in range(_G // _GI):
                    groups = [gg * _GI + q for q in range(_GI)]
                    ts = tuple(acc_v[g, t, :]
                               for g in groups for t in range(_K))

                    def jblock(jb, ts, groups=groups, b=b):
                        j = jb * _U
                        out_all = []
                        for q, g in enumerate(groups):
                            ts_g = list(ts[q * _K:(q + 1) * _K])
                            for u in range(_U):
                                cur = chunk_v[b, j + u, pl.ds(g * _L, _L)]
                                for t in range(_K):
                                    nt = jnp.maximum(ts_g[t], cur)
                                    cur = jnp.minimum(ts_g[t], cur)
                                    ts_g[t] = nt
                            out_all.extend(ts_g)
                        return tuple(out_all)

                    ts = lax.fori_loop(0, _CH // _U, jblock, ts)
                    for q, g in enumerate(groups):
                        for t in range(_K):
                            acc_v[g, t, :] = ts[q * _K + t]

                nci = ci + 2

                @pl.when(nci < _NCH)
                def _(b=b, nci=nci, src=src):
                    pltpu.async_copy(
                        src.at[pl.ds(nci * _CH, _CH), pl.ds(c0, _CW)],
                        chunk_v.at[b], sems[b])

        for g in range(_G):
            pos = pos_v[pl.ds(g * _L, _L)]
            s = jnp.exp(acc_v[g, 0, :] - pos)
            for t in range(1, _K):
                s = s + jnp.exp(acc_v[g, t, :] - pos)
            stage_v[pl.ds(g * _L, _L)] = s
        pltpu.sync_copy(stage_v, out_hbm.at[pl.ds(d * _B + c0, _CW)])


@functools.cache
def _topk_sc():
    return pl.kernel(
        _topk_body,
        out_type=jax.ShapeDtypeStruct((2 * _B,), jnp.float32),
        mesh=plsc.VectorSubcoreMesh(core_axis_name="c", subcore_axis_name="s",
                                    num_cores=_NC, num_subcores=_NS),
        scratch_types=[
            pltpu.VMEM((2, _CH, _CW), jnp.float32),
            pltpu.VMEM((_G, _K, _L), jnp.float32),
            pltpu.VMEM((_CW,), jnp.float32),
            pltpu.VMEM((_CW,), jnp.float32),
            pltpu.SemaphoreType.DMA,
            pltpu.SemaphoreType.DMA,
        ],
    )


def _reduce_body(s_ref, o_ref):
    o_ref[0, 0] = jnp.sum(jnp.log1p(s_ref[...])) * (1.0 / (2 * _B))


def _reduce(s):
    out = pl.pallas_call(
        _reduce_body,
        out_specs=pl.BlockSpec(memory_space=pltpu.SMEM),
        out_shape=jax.ShapeDtypeStruct((1, 1), jnp.float32),
    )(s.reshape(2 * _B // 128, 128))
    return out[0, 0]


def kernel(v_emb, a_emb):
    sim, simt, pos = _compute_sim(v_emb, a_emb)
    s = _topk_sc()(sim, simt, pos.reshape(_B))
    return _reduce(s)


# speedup vs baseline: 4.2491x; 1.1445x over previous
import functools

import jax
import jax.numpy as jnp
from jax import lax
from jax.experimental import pallas as pl
from jax.experimental.pallas import tpu as pltpu
from jax.experimental.pallas import tpu_sc as plsc

_TEMP = 0.07
_B = 4096
_D = 16
_K = 5
_NC, _NS, _L = 2, 16, 16
_NW = _NC * _NS
_CW = 256
_CH = 128
_NCH = _B // _CH
_NG = _CW // _L
_GI = 4
_RB = 256
_NEG = -3.0e38


def _norm_rows(x):
    return x * lax.rsqrt(jnp.maximum(jnp.sum(x * x, axis=1, keepdims=True),
                                     1e-24))


def _sim_body(v_blk_ref, a_blk_ref, v_all_ref, a_all_ref, m_ref, pos_ref):
    i = pl.program_id(0)
    vb = _norm_rows(v_blk_ref[...])
    ab = _norm_rows(a_blk_ref[...])
    va = _norm_rows(v_all_ref[...])
    aa = _norm_rows(a_all_ref[...])

    def mm(x, y):
        return lax.dot_general(x, y, (((1,), (1,)), ((), ())),
                               preferred_element_type=jnp.float32) * (1.0 / _TEMP)

    row_ids = (i % 16) * _RB + lax.broadcasted_iota(jnp.int32, (_RB, _B), 0)
    col_ids = lax.broadcasted_iota(jnp.int32, (_RB, _B), 1)
    blk = jnp.where(pl.program_id(0) < 16, mm(ab, va), mm(vb, aa))
    m_ref[...] = jnp.where(row_ids == col_ids, _NEG, blk).astype(jnp.bfloat16)
    pos_ref[...] = (jnp.sum(vb * ab, axis=1) * (1.0 / _TEMP)).reshape(1, 1, _RB)


def _compute_sim(v_emb, a_emb):
    full = pl.BlockSpec((_B, _D), lambda i: (0, 0))
    return pl.pallas_call(
        _sim_body,
        grid=(2 * _B // _RB,),
        in_specs=[
            pl.BlockSpec((_RB, _D), lambda i: (i % 16, 0)),
            pl.BlockSpec((_RB, _D), lambda i: (i % 16, 0)),
            full, full,
        ],
        out_specs=[
            pl.BlockSpec((_RB, _B), lambda i: (i, 0)),
            pl.BlockSpec((1, 1, _RB), lambda i: (i % 16, 0, 0)),
        ],
        out_shape=[
            jax.ShapeDtypeStruct((2 * _B, _B), jnp.bfloat16),
            jax.ShapeDtypeStruct((_B // _RB, 1, _RB), jnp.float32),
        ],
    )(v_emb, a_emb, v_emb, a_emb)


def _topk_body(m_hbm, out_hbm, chunk_v, acc_v, stage_v, sem0, sem1):
    wid = lax.axis_index("s") * _NC + lax.axis_index("c")
    dd = wid & 1
    w2 = wid >> 1
    cb = w2 * _CW
    rbase = dd * _B
    neg = jnp.full((2, _L), _NEG, jnp.bfloat16)
    sems = (sem0, sem1)

    for g in range(_NG):
        for t in range(_K):
            acc_v[g, t, :, :] = neg

    for b in range(2):
        pltpu.async_copy(
            m_hbm.at[pl.ds(rbase + b * _CH, _CH), pl.ds(cb, _CW)],
            chunk_v.at[b], sems[b])

    @pl.loop(0, _NCH, step=2)
    def _(ci0):
        for b in range(2):
            ci = ci0 + b
            pltpu.make_async_copy(
                m_hbm.at[pl.ds(0, _CH), pl.ds(cb, _CW)],
                chunk_v.at[b], sems[b]).wait()
            for gg in range(_NG // _GI):
                groups = [gg * _GI + q for q in range(_GI)]
                ts = tuple(acc_v[g, t, :, :]
                           for g in groups for t in range(_K))

                def jblock(jj, ts, groups=groups, b=b):
                    j = pl.multiple_of(2 * jj, 2)
                    out_all = []
                    for q, g in enumerate(groups):
                        cur = chunk_v[b, pl.ds(j, 2), pl.ds(g * _L, _L)]
                        ts_g = list(ts[q * _K:(q + 1) * _K])
                        for t in range(_K):
                            nt = jnp.maximum(ts_g[t], cur)
                            cur = jnp.minimum(ts_g[t], cur)
                            ts_g[t] = nt
                        out_all.extend(ts_g)
                    return tuple(out_all)

                ts = lax.fori_loop(0, _CH // 2, jblock, ts)
                for q, g in enumerate(groups):
                    for t in range(_K):
                        acc_v[g, t, :, :] = ts[q * _K + t]

            nci = ci + 2

            @pl.when(nci < _NCH)
            def _(b=b, nci=nci):
                pltpu.async_copy(
                    m_hbm.at[pl.ds(rbase + nci * _CH, _CH), pl.ds(cb, _CW)],
                    chunk_v.at[b], sems[b])

    for t in range(_K):
        for g in range(_NG):
            stage_v[t, :, pl.ds(g * _L, _L)] = acc_v[g, t, :, :]
    pltpu.sync_copy(stage_v, out_hbm.at[dd * (_NW // 2) + w2])


@functools.cache
def _topk_sc():
    return pl.kernel(
        _topk_body,
        out_type=jax.ShapeDtypeStruct((_NW, _K, 2, _CW), jnp.bfloat16),
        mesh=plsc.VectorSubcoreMesh(core_axis_name="c", subcore_axis_name="s",
                                    num_cores=_NC, num_subcores=_NS),
        scratch_types=[
            pltpu.VMEM((2, _CH, _CW), jnp.bfloat16),
            pltpu.VMEM((_NG, _K, 2, _L), jnp.bfloat16),
            pltpu.VMEM((_K, 2, _CW), jnp.bfloat16),
            pltpu.SemaphoreType.DMA,
            pltpu.SemaphoreType.DMA,
        ],
    )


def _reduce_body(s_ref, pos_ref, o_ref):
    x = s_ref[...].astype(jnp.float32)
    p = pos_ref[...]
    s = jnp.zeros((_NW, 1, _CW), jnp.float32)
    ii = lax.broadcasted_iota(jnp.int32, x.shape, 1)
    for _ in range(_K):
        mx = jnp.max(x, axis=1, keepdims=True)
        is_mx = x == mx
        mn_i = jnp.min(jnp.where(is_mx, ii, 2 * _K), axis=1, keepdims=True)
        s = s + jnp.exp(mx - p)
        x = jnp.where(ii == mn_i, _NEG, x)
    o_ref[0, 0] = jnp.sum(jnp.log1p(s)) * (1.0 / (2 * _B))


def _reduce(s10, pos3):
    out = pl.pallas_call(
        _reduce_body,
        out_specs=pl.BlockSpec(memory_space=pltpu.SMEM),
        out_shape=jax.ShapeDtypeStruct((1, 1), jnp.float32),
    )(s10, pos3)
    return out[0, 0]


def kernel(v_emb, a_emb):
    m, pos = _compute_sim(v_emb, a_emb)
    s10 = _topk_sc()(m).reshape(_NW, 2 * _K, _CW)
    pos3 = jnp.tile(pos.reshape(_NW // 2, 1, _CW), (2, 1, 1))
    return _reduce(s10, pos3)


# speedup vs baseline: 5.0529x; 1.1891x over previous
import functools

import jax
import jax.numpy as jnp
from jax import lax
from jax.experimental import pallas as pl
from jax.experimental.pallas import tpu as pltpu
from jax.experimental.pallas import tpu_sc as plsc

_TEMP = 0.07
_B = 4096
_D = 16
_K = 5
_NC, _NS, _L = 2, 16, 16
_NW = _NC * _NS
_CW = 256
_CH = 128
_NCH = _B // _CH
_NG = _CW // _L
_GI = 4
_RB = 256
_NEG = -3.0e38


def _norm_rows(x):
    return x * lax.rsqrt(jnp.maximum(jnp.sum(x * x, axis=1, keepdims=True),
                                     1e-24))


def _norm_body(v_ref, a_ref, vn_ref, an_ref, pos_ref):
    vn = _norm_rows(v_ref[...])
    an = _norm_rows(a_ref[...])
    vn_ref[...] = vn.astype(jnp.bfloat16)
    an_ref[...] = an.astype(jnp.bfloat16)
    pos_ref[...] = jnp.sum(vn * an, axis=1, keepdims=True) * (1.0 / _TEMP)


def _normalize(v_emb, a_emb):
    return pl.pallas_call(
        _norm_body,
        out_shape=[
            jax.ShapeDtypeStruct((_B, _D), jnp.bfloat16),
            jax.ShapeDtypeStruct((_B, _D), jnp.bfloat16),
            jax.ShapeDtypeStruct((_B, 1), jnp.float32),
        ],
    )(v_emb, a_emb)


def _sim_body(lhs_ref, rhs_ref, m_ref):
    i = pl.program_id(0)
    blk = lax.dot_general(lhs_ref[...], rhs_ref[0], (((1,), (1,)), ((), ())),
                          preferred_element_type=jnp.float32) * (1.0 / _TEMP)
    row_ids = (i % 16) * _RB + lax.broadcasted_iota(jnp.int32, (_RB, _B), 0)
    col_ids = lax.broadcasted_iota(jnp.int32, (_RB, _B), 1)
    m_ref[...] = jnp.where(row_ids == col_ids, _NEG, blk).astype(jnp.bfloat16)


def _compute_sim(lhs_cat, rhs_cat):
    return pl.pallas_call(
        _sim_body,
        grid=(2 * _B // _RB,),
        in_specs=[
            pl.BlockSpec((_RB, _D), lambda i: (i, 0)),
            pl.BlockSpec((1, _B, _D), lambda i: (i // 16, 0, 0)),
        ],
        out_specs=pl.BlockSpec((_RB, _B), lambda i: (i, 0)),
        out_shape=jax.ShapeDtypeStruct((2 * _B, _B), jnp.bfloat16),
    )(lhs_cat, rhs_cat)


def _topk_body(m_hbm, out_hbm, chunk_v, acc_v, stage_v, sem0, sem1):
    wid = lax.axis_index("s") * _NC + lax.axis_index("c")
    dd = wid & 1
    w2 = wid >> 1
    cb = w2 * _CW
    rbase = dd * _B
    neg = jnp.full((2, _L), _NEG, jnp.bfloat16)
    sems = (sem0, sem1)

    for g in range(_NG):
        for t in range(_K):
            acc_v[g, t, :, :] = neg

    for b in range(2):
        pltpu.async_copy(
            m_hbm.at[pl.ds(rbase + b * _CH, _CH), pl.ds(cb, _CW)],
            chunk_v.at[b], sems[b])

    @pl.loop(0, _NCH, step=2)
    def _(ci0):
        for b in range(2):
            ci = ci0 + b
            pltpu.make_async_copy(
                m_hbm.at[pl.ds(0, _CH), pl.ds(cb, _CW)],
                chunk_v.at[b], sems[b]).wait()
            for gg in range(_NG // _GI):
                groups = [gg * _GI + q for q in range(_GI)]
                ts = tuple(acc_v[g, t, :, :]
                           for g in groups for t in range(_K))

                def jblock(jj, ts, groups=groups, b=b):
                    j = pl.multiple_of(2 * jj, 2)
                    out_all = []
                    for q, g in enumerate(groups):
                        cur = chunk_v[b, pl.ds(j, 2), pl.ds(g * _L, _L)]
                        ts_g = list(ts[q * _K:(q + 1) * _K])
                        for t in range(_K):
                            nt = jnp.maximum(ts_g[t], cur)
                            cur = jnp.minimum(ts_g[t], cur)
                            ts_g[t] = nt
                        out_all.extend(ts_g)
                    return tuple(out_all)

                ts = lax.fori_loop(0, _CH // 2, jblock, ts)
                for q, g in enumerate(groups):
                    for t in range(_K):
                        acc_v[g, t, :, :] = ts[q * _K + t]

            nci = ci + 2

            @pl.when(nci < _NCH)
            def _(b=b, nci=nci):
                pltpu.async_copy(
                    m_hbm.at[pl.ds(rbase + nci * _CH, _CH), pl.ds(cb, _CW)],
                    chunk_v.at[b], sems[b])

    for t in range(_K):
        for g in range(_NG):
            stage_v[t, :, pl.ds(g * _L, _L)] = acc_v[g, t, :, :]
    pltpu.sync_copy(stage_v, out_hbm.at[dd * (_NW // 2) + w2])


@functools.cache
def _topk_sc():
    return pl.kernel(
        _topk_body,
        out_type=jax.ShapeDtypeStruct((_NW, _K, 2, _CW), jnp.bfloat16),
        mesh=plsc.VectorSubcoreMesh(core_axis_name="c", subcore_axis_name="s",
                                    num_cores=_NC, num_subcores=_NS),
        scratch_types=[
            pltpu.VMEM((2, _CH, _CW), jnp.bfloat16),
            pltpu.VMEM((_NG, _K, 2, _L), jnp.bfloat16),
            pltpu.VMEM((_K, 2, _CW), jnp.bfloat16),
            pltpu.SemaphoreType.DMA,
            pltpu.SemaphoreType.DMA,
        ],
    )


def _reduce_body(s_ref, pos_ref, o_ref):
    x = s_ref[...].astype(jnp.float32)
    p16 = pos_ref[...]
    p = jnp.concatenate([p16, p16], axis=0)
    s = jnp.zeros((_NW, 1, _CW), jnp.float32)
    ii = lax.broadcasted_iota(jnp.int32, x.shape, 1)
    for _ in range(_K):
        mx = jnp.max(x, axis=1, keepdims=True)
        is_mx = x == mx
        mn_i = jnp.min(jnp.where(is_mx, ii, 2 * _K), axis=1, keepdims=True)
        s = s + jnp.exp(mx - p)
        x = jnp.where(ii == mn_i, _NEG, x)
    o_ref[0, 0] = jnp.sum(jnp.log1p(s)) * (1.0 / (2 * _B))


def _reduce(s10, pos3):
    out = pl.pallas_call(
        _reduce_body,
        out_specs=pl.BlockSpec(memory_space=pltpu.SMEM),
        out_shape=jax.ShapeDtypeStruct((1, 1), jnp.float32),
    )(s10, pos3)
    return out[0, 0]


def kernel(v_emb, a_emb):
    vn, an, pos = _normalize(v_emb, a_emb)
    lhs_cat = jnp.concatenate([an, vn], axis=0)
    rhs_cat = jnp.stack([vn, an], axis=0)
    m = _compute_sim(lhs_cat, rhs_cat)
    s10 = _topk_sc()(m).reshape(_NW, 2 * _K, _CW)
    return _reduce(s10, pos.reshape(_NW // 2, 1, _CW))


# speedup vs baseline: 5.6467x; 1.1175x over previous
import functools

import jax
import jax.numpy as jnp
from jax import lax
from jax.experimental import pallas as pl
from jax.experimental.pallas import tpu as pltpu
from jax.experimental.pallas import tpu_sc as plsc

_TEMP = 0.07
_B = 4096
_D = 16
_K = 5
_NC, _NS, _L = 2, 16, 16
_NW = _NC * _NS
_CW = _B // _NW
_CH = 128
_NCH = _B // _CH
_NG = _CW // _L
_GI = 4
_RB = 256
_NEG = -3.0e38


def _norm_rows(x):
    return x * lax.rsqrt(jnp.maximum(jnp.sum(x * x, axis=1, keepdims=True),
                                     1e-24))


def _norm_body(v_ref, a_ref, vn_ref, an_ref, pos_ref):
    vn = _norm_rows(v_ref[...])
    an = _norm_rows(a_ref[...])
    vn_ref[...] = vn.astype(jnp.bfloat16)
    an_ref[...] = an.astype(jnp.bfloat16)
    pos_ref[...] = jnp.sum(vn * an, axis=1, keepdims=True) * (1.0 / _TEMP)


def _normalize(v_emb, a_emb):
    return pl.pallas_call(
        _norm_body,
        out_shape=[
            jax.ShapeDtypeStruct((_B, _D), jnp.bfloat16),
            jax.ShapeDtypeStruct((_B, _D), jnp.bfloat16),
            jax.ShapeDtypeStruct((_B, 1), jnp.float32),
        ],
    )(v_emb, a_emb)


def _sim_body(lhs_ref, rhs_ref, m_ref):
    i = pl.program_id(0)
    blk = lax.dot_general(lhs_ref[...], rhs_ref[...], (((1,), (1,)), ((), ())),
                          preferred_element_type=jnp.float32) * (1.0 / _TEMP)
    row_ids = i * _RB + lax.broadcasted_iota(jnp.int32, (_RB, _B), 0)
    col_ids = lax.broadcasted_iota(jnp.int32, (_RB, _B), 1)
    m_ref[...] = jnp.where(row_ids == col_ids, _NEG, blk).astype(jnp.bfloat16)


def _compute_sim(lhs, rhs):
    return pl.pallas_call(
        _sim_body,
        grid=(_B // _RB,),
        in_specs=[
            pl.BlockSpec((_RB, _D), lambda i: (i, 0)),
            pl.BlockSpec((_B, _D), lambda i: (0, 0)),
        ],
        out_specs=pl.BlockSpec((_RB, _B), lambda i: (i, 0)),
        out_shape=jax.ShapeDtypeStruct((_B, _B), jnp.bfloat16),
    )(lhs, rhs)


def _topk_body(m_hbm, out_hbm, chunk_v, acc_v, stage_v, sem0, sem1):
    wid = lax.axis_index("s") * _NC + lax.axis_index("c")
    cb = wid * _CW
    neg = jnp.full((2, _L), _NEG, jnp.bfloat16)
    sems = (sem0, sem1)

    for g in range(_NG):
        for t in range(_K):
            acc_v[g, t, :, :] = neg

    for b in range(2):
        pltpu.async_copy(
            m_hbm.at[pl.ds(b * _CH, _CH), pl.ds(cb, _CW)],
            chunk_v.at[b], sems[b])

    @pl.loop(0, _NCH, step=2)
    def _(ci0):
        for b in range(2):
            ci = ci0 + b
            pltpu.make_async_copy(
                m_hbm.at[pl.ds(0, _CH), pl.ds(cb, _CW)],
                chunk_v.at[b], sems[b]).wait()
            for gg in range(_NG // _GI):
                groups = [gg * _GI + q for q in range(_GI)]
                ts = tuple(acc_v[g, t, :, :]
                           for g in groups for t in range(_K))

                def jblock(jj, ts, groups=groups, b=b):
                    j = pl.multiple_of(2 * jj, 2)
                    out_all = []
                    for q, g in enumerate(groups):
                        cur = chunk_v[b, pl.ds(j, 2), pl.ds(g * _L, _L)]
                        ts_g = list(ts[q * _K:(q + 1) * _K])
                        for t in range(_K):
                            nt = jnp.maximum(ts_g[t], cur)
                            cur = jnp.minimum(ts_g[t], cur)
                            ts_g[t] = nt
                        out_all.extend(ts_g)
                    return tuple(out_all)

                ts = lax.fori_loop(0, _CH // 2, jblock, ts)
                for q, g in enumerate(groups):
                    for t in range(_K):
                        acc_v[g, t, :, :] = ts[q * _K + t]

            nci = ci + 2

            @pl.when(nci < _NCH)
            def _(b=b, nci=nci):
                pltpu.async_copy(
                    m_hbm.at[pl.ds(nci * _CH, _CH), pl.ds(cb, _CW)],
                    chunk_v.at[b], sems[b])

    for t in range(_K):
        for g in range(_NG):
            stage_v[t, :, pl.ds(g * _L, _L)] = acc_v[g, t, :, :]
    pltpu.sync_copy(stage_v, out_hbm.at[wid])


@functools.cache
def _topk_sc():
    return pl.kernel(
        _topk_body,
        out_type=jax.ShapeDtypeStruct((_NW, _K, 2, _CW), jnp.bfloat16),
        mesh=plsc.VectorSubcoreMesh(core_axis_name="c", subcore_axis_name="s",
                                    num_cores=_NC, num_subcores=_NS),
        scratch_types=[
            pltpu.VMEM((2, _CH, _CW), jnp.bfloat16),
            pltpu.VMEM((_NG, _K, 2, _L), jnp.bfloat16),
            pltpu.VMEM((_K, 2, _CW), jnp.bfloat16),
            pltpu.SemaphoreType.DMA,
            pltpu.SemaphoreType.DMA,
        ],
    )


def _reduce_body(s_ref, pos_ref, o_ref):
    x = s_ref[...].astype(jnp.float32)
    p32 = pos_ref[...]
    p = jnp.concatenate([p32, p32], axis=0)
    s = jnp.zeros((2 * _NW, 1, _CW), jnp.float32)
    ii = lax.broadcasted_iota(jnp.int32, x.shape, 1)
    for _ in range(_K):
        mx = jnp.max(x, axis=1, keepdims=True)
        is_mx = x == mx
        mn_i = jnp.min(jnp.where(is_mx, ii, 2 * _K), axis=1, keepdims=True)
        s = s + jnp.exp(mx - p)
        x = jnp.where(ii == mn_i, _NEG, x)
    o_ref[0, 0] = jnp.sum(jnp.log1p(s)) * (1.0 / (2 * _B))


def _reduce(s10, pos3):
    out = pl.pallas_call(
        _reduce_body,
        out_specs=pl.BlockSpec(memory_space=pltpu.SMEM),
        out_shape=jax.ShapeDtypeStruct((1, 1), jnp.float32),
    )(s10, pos3)
    return out[0, 0]


def kernel(v_emb, a_emb):
    vn, an, pos = _normalize(v_emb, a_emb)
    mt = _compute_sim(an, vn)
    s10_t = _topk_sc()(mt)
    ms = _compute_sim(vn, an)
    s10_s = _topk_sc()(ms)
    s10 = jnp.concatenate([s10_t, s10_s], axis=0).reshape(2 * _NW, 2 * _K, _CW)
    return _reduce(s10, pos.reshape(_NW, 1, _CW))
